# baseline (speedup 1.0000x reference)
import functools

import jax
import jax.numpy as jnp
from jax import lax
from jax.experimental import pallas as pl
from jax.experimental.pallas import tpu as pltpu
from jax.experimental.pallas import tpu_sc as plsc

NB = 256


def _gelu(x):
    return 0.5 * x * (1.0 + lax.erf(x * 0.7071067811865476))


def _pack_bf16_pair(q):
    h = q.shape[-1] // 2
    a = jax.lax.bitcast_convert_type(q[:, :h].astype(jnp.bfloat16), jnp.uint16)
    b = jax.lax.bitcast_convert_type(q[:, h:].astype(jnp.bfloat16), jnp.uint16)
    w = a.astype(jnp.uint32) | (b.astype(jnp.uint32) << 16)
    return jax.lax.bitcast_convert_type(w, jnp.int32)


def _unpack_bf16_pair(g):
    gu = jax.lax.bitcast_convert_type(g, jnp.uint32)
    a = jax.lax.bitcast_convert_type((gu & 0xFFFF).astype(jnp.uint16), jnp.bfloat16)
    b = jax.lax.bitcast_convert_type((gu >> 16).astype(jnp.uint16), jnp.bfloat16)
    return jnp.concatenate([a.astype(jnp.float32), b.astype(jnp.float32)], axis=-1)


def _ln(x, g, b):
    m = jnp.mean(x, axis=-1, keepdims=True)
    c = x - m
    v = jnp.mean(c * c, axis=-1, keepdims=True)
    return c * jax.lax.rsqrt(v + 1e-5) * g + b


def _stage_a_body(v_ref, wi_ref, bi_ref, wj_ref, p_ref, q_ref):
    v = v_ref[...].astype(jnp.bfloat16)
    p_ref[...] = jnp.dot(v, wi_ref[...], preferred_element_type=jnp.float32) + bi_ref[...]
    q = jnp.dot(v, wj_ref[...], preferred_element_type=jnp.float32)
    q_ref[...] = _pack_bf16_pair(q)


def _stage_a(v_flat, w1_i, b1, w1_j):
    zn, d = v_flat.shape
    return pl.pallas_call(
        _stage_a_body,
        out_shape=(jax.ShapeDtypeStruct((zn, d), jnp.float32),
                   jax.ShapeDtypeStruct((zn, d // 2), jnp.int32)),
    )(v_flat, w1_i, b1.reshape(1, d), w1_j)


def _sc_gather(table, idx):
    bsz = idx.shape[0]
    d = table.shape[1]
    info = plsc.get_sparse_core_info()
    nw = info.num_cores * info.num_subcores
    b_per_w = bsz // nw
    ch = 128
    n_ch = b_per_w // ch
    nbuf = 3
    mesh = plsc.VectorSubcoreMesh(core_axis_name="c", subcore_axis_name="s")

    @functools.partial(
        pl.kernel,
        mesh=mesh,
        out_type=jax.ShapeDtypeStruct((bsz, d), jnp.int32),
        scratch_types=[
            pltpu.VMEM((b_per_w,), jnp.int32),
            *[pltpu.VMEM((ch, d), jnp.int32) for _ in range(nbuf)],
            *[pltpu.SemaphoreType.DMA for _ in range(2 * nbuf)],
        ],
    )
    def gather_k(table_hbm, idx_hbm, out_hbm, idx_v, *bufs_and_sems):
        rows = bufs_and_sems[:nbuf]
        gsem = bufs_and_sems[nbuf:2 * nbuf]
        osem = bufs_and_sems[2 * nbuf:]
        wid = lax.axis_index("s") * info.num_cores + lax.axis_index("c")
        base = wid * b_per_w
        pltpu.sync_copy(idx_hbm.at[pl.ds(base, b_per_w)], idx_v)

        gcp = [None] * n_ch
        ocp = [None] * n_ch
        for c in range(n_ch):
            b = c % nbuf
            if c >= nbuf:
                ocp[c - nbuf].wait()
            gcp[c] = pltpu.async_copy(
                table_hbm.at[idx_v.at[pl.ds(c * ch, ch)]], rows[b], gsem[b])
            if c >= 1:
                pb = (c - 1) % nbuf
                gcp[c - 1].wait()
                ocp[c - 1] = pltpu.async_copy(
                    rows[pb], out_hbm.at[pl.ds(base + (c - 1) * ch, ch)], osem[pb])
        lb = (n_ch - 1) % nbuf
        gcp[n_ch - 1].wait()
        ocp[n_ch - 1] = pltpu.async_copy(
            rows[lb], out_hbm.at[pl.ds(base + (n_ch - 1) * ch, ch)], osem[lb])
        for c in range(max(0, n_ch - nbuf), n_ch):
            ocp[c].wait()

    return gather_k(table, idx)


def _stage_b_body(v_ref, p1_ref, g1_ref, e_ref, mask_ref,
                  w1e_ref, w2_ref, b2_ref, w3_ref, b3_ref,
                  nmg_ref, nmb_ref,
                  fw1_ref, fb1_ref, fw2_ref, fb2_ref, fg_ref, fb_ref,
                  ewi_ref, ebi_ref, ewj_ref,
                  v2_ref, p2_ref, q2_ref):
    nb, k = g1_ref.shape[1], g1_ref.shape[2]
    d = e_ref.shape[3]
    e = e_ref[0].reshape(nb * k, d).astype(jnp.bfloat16)
    g1 = _unpack_bf16_pair(g1_ref[0].reshape(nb * k, d // 2))
    p1 = p1_ref[0]
    x = jnp.dot(e, w1e_ref[...], preferred_element_type=jnp.float32) + g1
    x = (x.reshape(nb, k, d) + p1[:, None, :]).reshape(nb * k, d)
    h = _gelu(x).astype(jnp.bfloat16)
    h = _gelu(jnp.dot(h, w2_ref[...], preferred_element_type=jnp.float32) + b2_ref[...]).astype(jnp.bfloat16)
    m = jnp.dot(h, w3_ref[...], preferred_element_type=jnp.float32) + b3_ref[...]
    m = m.reshape(nb, k, d) * mask_ref[0][:, :, None]
    msum = jnp.sum(m, axis=1)
    v1 = _ln(v_ref[0] + msum, nmg_ref[...], nmb_ref[...])
    f = _gelu(jnp.dot(v1.astype(jnp.bfloat16), fw1_ref[...], preferred_element_type=jnp.float32) + fb1_ref[...]).astype(jnp.bfloat16)
    f = jnp.dot(f, fw2_ref[...], preferred_element_type=jnp.float32) + fb2_ref[...]
    v2 = _ln(v1 + f, fg_ref[...], fb_ref[...])
    v2_ref[0] = v2
    v2b = v2.astype(jnp.bfloat16)
    p2_ref[0] = jnp.dot(v2b, ewi_ref[...], preferred_element_type=jnp.float32) + ebi_ref[...]
    q2_ref[0] = _pack_bf16_pair(jnp.dot(v2b, ewj_ref[...], preferred_element_type=jnp.float32))


def _stage_b(zi, V, P1, G1z, E, mask, w1e, w2, b2, w3, b3, nmg, nmb,
             fw1, fb1, fw2, fb2, fg, fb, ewi, ebi, ewj):
    z, n, d = V.shape
    k = E.shape[2]
    grid = (n // NB,)
    node_spec = pl.BlockSpec((1, NB, d), lambda ni: (zi, ni, 0))
    out_node_spec = pl.BlockSpec((1, NB, d), lambda ni: (0, ni, 0))
    out_q_spec = pl.BlockSpec((1, NB, d // 2), lambda ni: (0, ni, 0))
    g_spec = pl.BlockSpec((1, NB, k, d // 2), lambda ni: (0, ni, 0, 0))
    edge_spec = pl.BlockSpec((1, NB, k, d), lambda ni: (zi, ni, 0, 0))
    mask_spec = pl.BlockSpec((1, NB, k), lambda ni: (zi, ni, 0))

    def wspec(shape):
        return pl.BlockSpec(shape, lambda ni: tuple(0 for _ in shape))

    out = jax.ShapeDtypeStruct((1, n, d), jnp.float32)
    out_q = jax.ShapeDtypeStruct((1, n, d // 2), jnp.int32)
    return pl.pallas_call(
        _stage_b_body,
        grid=grid,
        in_specs=[
            node_spec, node_spec, g_spec, edge_spec, mask_spec,
            wspec((d, d)), wspec((d, d)), wspec((1, d)), wspec((d, d)), wspec((1, d)),
            wspec((1, d)), wspec((1, d)),
            wspec((d, 4 * d)), wspec((1, 4 * d)), wspec((4 * d, d)), wspec((1, d)),
            wspec((1, d)), wspec((1, d)),
            wspec((d, d)), wspec((1, d)), wspec((d, d)),
        ],
        out_specs=(out_node_spec, out_node_spec, out_q_spec),
        out_shape=(out, out, out_q),
    )(V, P1, G1z, E, mask, w1e, w2, b2, w3, b3, nmg, nmb,
      fw1, fb1, fw2, fb2, fg, fb, ewi, ebi, ewj)


def _stage_c_body(p2_ref, g2_ref, e_ref, mask_ref,
                  w1e_ref, w2_ref, b2_ref, w3_ref, b3_ref,
                  lg_ref, lb_ref, *chain_and_out):
    eout_ref = chain_and_out[-1]
    nb, k = g2_ref.shape[1], g2_ref.shape[2]
    d = e_ref.shape[3]
    e = e_ref[0].reshape(nb * k, d)
    g2 = _unpack_bf16_pair(g2_ref[0].reshape(nb * k, d // 2))
    p2 = p2_ref[0]
    x = jnp.dot(e.astype(jnp.bfloat16), w1e_ref[...], preferred_element_type=jnp.float32) + g2
    x = (x.reshape(nb, k, d) + p2[:, None, :]).reshape(nb * k, d)
    h = _gelu(x).astype(jnp.bfloat16)
    h = _gelu(jnp.dot(h, w2_ref[...], preferred_element_type=jnp.float32) + b2_ref[...]).astype(jnp.bfloat16)
    m = jnp.dot(h, w3_ref[...], preferred_element_type=jnp.float32) + b3_ref[...]
    m = m.reshape(nb, k, d) * mask_ref[0][:, :, None]
    eout_ref[0] = _ln(e.reshape(nb, k, d) + m, lg_ref[...], lb_ref[...])


def _stage_c(zi, chain, P2z, G2z, E, mask, w1e, w2, b2, w3, b3, lg, lb):
    z, n, k, d = E.shape
    grid = (n // NB,)
    node_spec = pl.BlockSpec((1, NB, d), lambda ni: (0, ni, 0))
    g_spec = pl.BlockSpec((1, NB, k, d // 2), lambda ni: (0, ni, 0, 0))
    edge_spec = pl.BlockSpec((1, NB, k, d), lambda ni: (zi, ni, 0, 0))
    mask_spec = pl.BlockSpec((1, NB, k), lambda ni: (zi, ni, 0))

    def wspec(shape):
        return pl.BlockSpec(shape, lambda ni: tuple(0 for _ in shape))

    in_specs = [
        node_spec, g_spec, edge_spec, mask_spec,
        wspec((d, d)), wspec((d, d)), wspec((1, d)), wspec((d, d)), wspec((1, d)),
        wspec((1, d)), wspec((1, d)),
    ]
    args = [P2z, G2z, E, mask, w1e, w2, b2, w3, b3, lg, lb]
    aliases = {}
    if chain is not None:
        in_specs.append(edge_spec)
        args.append(chain)
        aliases = {11: 0}
    return pl.pallas_call(
        _stage_c_body,
        grid=grid,
        in_specs=in_specs,
        out_specs=edge_spec,
        out_shape=jax.ShapeDtypeStruct((z, n, k, d), jnp.float32),
        input_output_aliases=aliases,
    )(*args)


def kernel(V, E, K, edge_mask, nm_W1, nm_b1, nm_W2, nm_b2, nm_W3, nm_b3,
           nm_ln_g, nm_ln_b, ffn_W1, ffn_b1, ffn_W2, ffn_b2, ffn_ln_g, ffn_ln_b,
           em_W1, em_b1, em_W2, em_b2, em_W3, em_b3, em_ln_g, em_ln_b):
    z, n, d = V.shape
    k = K.shape[2]

    bf = jnp.bfloat16
    nm_w1_i, nm_w1_j, nm_w1_e = (nm_W1[:d].astype(bf), nm_W1[d:2 * d].astype(bf),
                                 nm_W1[2 * d:].astype(bf))
    em_w1_i, em_w1_j, em_w1_e = (em_W1[:d].astype(bf), em_W1[d:2 * d].astype(bf),
                                 em_W1[2 * d:].astype(bf))
    nm_W2, nm_W3 = nm_W2.astype(bf), nm_W3.astype(bf)
    em_W2, em_W3 = em_W2.astype(bf), em_W3.astype(bf)
    ffn_W1, ffn_W2 = ffn_W1.astype(bf), ffn_W2.astype(bf)

    idx_z = K.reshape(z, n * k)

    v_flat = V.reshape(z * n, d)
    P1, Q1 = _stage_a(v_flat, nm_w1_i, nm_b1, nm_w1_j)
    P1 = P1.reshape(z, n, d)
    Q1 = Q1.reshape(z, n, d // 2)

    b_args = (nm_w1_e, nm_W2, nm_b2.reshape(1, d), nm_W3, nm_b3.reshape(1, d),
              nm_ln_g.reshape(1, d), nm_ln_b.reshape(1, d),
              ffn_W1, ffn_b1.reshape(1, 4 * d), ffn_W2, ffn_b2.reshape(1, d),
              ffn_ln_g.reshape(1, d), ffn_ln_b.reshape(1, d),
              em_w1_i, em_b1.reshape(1, d), em_w1_j)
    c_args = (em_w1_e, em_W2, em_b2.reshape(1, d), em_W3, em_b3.reshape(1, d),
              em_ln_g.reshape(1, d), em_ln_b.reshape(1, d))

    G1 = [None] * z
    for zi in range(z):
        G1[zi] = _sc_gather(Q1[zi], idx_z[zi]).reshape(1, n, k, d // 2)

    V2 = [None] * z
    P2 = [None] * z
    Q2 = [None] * z
    for zi in range(z):
        V2[zi], P2[zi], Q2[zi] = _stage_b(
            zi, V, P1, G1[zi], E, edge_mask, *b_args)

    G2 = [None] * z
    for zi in range(z):
        G2[zi] = _sc_gather(Q2[zi].reshape(n, d // 2), idx_z[zi]).reshape(1, n, k, d // 2)

    chain = None
    for zi in range(z):
        chain = _stage_c(zi, chain, P2[zi], G2[zi], E, edge_mask, *c_args)

    return (jnp.concatenate(V2, axis=0), chain)


# speedup vs baseline: 1.0310x; 1.0310x over previous
import functools

import jax
import jax.numpy as jnp
from jax import lax
from jax.experimental import pallas as pl
from jax.experimental.pallas import tpu as pltpu
from jax.experimental.pallas import tpu_sc as plsc

NB = 256


def _gelu(x):
    return 0.5 * x * (1.0 + lax.erf(x * 0.7071067811865476))


def _pack_bf16_pair(q):
    h = q.shape[-1] // 2
    a = jax.lax.bitcast_convert_type(q[:, :h].astype(jnp.bfloat16), jnp.uint16)
    b = jax.lax.bitcast_convert_type(q[:, h:].astype(jnp.bfloat16), jnp.uint16)
    w = a.astype(jnp.uint32) | (b.astype(jnp.uint32) << 16)
    return jax.lax.bitcast_convert_type(w, jnp.int32)


def _unpack_bf16_pair(g):
    gu = jax.lax.bitcast_convert_type(g, jnp.uint32)
    a = jax.lax.bitcast_convert_type((gu & 0xFFFF).astype(jnp.uint16), jnp.bfloat16)
    b = jax.lax.bitcast_convert_type((gu >> 16).astype(jnp.uint16), jnp.bfloat16)
    return jnp.concatenate([a.astype(jnp.float32), b.astype(jnp.float32)], axis=-1)


def _ln(x, g, b):
    m = jnp.mean(x, axis=-1, keepdims=True)
    c = x - m
    v = jnp.mean(c * c, axis=-1, keepdims=True)
    return c * jax.lax.rsqrt(v + 1e-5) * g + b


def _stage_a_body(v_ref, wi_ref, bi_ref, wj_ref, p_ref, q_ref):
    v = v_ref[...].astype(jnp.bfloat16)
    p_ref[...] = jnp.dot(v, wi_ref[...], preferred_element_type=jnp.float32) + bi_ref[...]
    q = jnp.dot(v, wj_ref[...], preferred_element_type=jnp.float32)
    q_ref[...] = _pack_bf16_pair(q)


def _stage_a(v_flat, w1_i, b1, w1_j):
    zn, d = v_flat.shape
    return pl.pallas_call(
        _stage_a_body,
        out_shape=(jax.ShapeDtypeStruct((zn, d), jnp.float32),
                   jax.ShapeDtypeStruct((zn, d // 2), jnp.int32)),
    )(v_flat, w1_i, b1.reshape(1, d), w1_j)


def _sc_gather(table, idx):
    bsz = idx.shape[0]
    d = table.shape[1]
    info = plsc.get_sparse_core_info()
    nw = info.num_cores * info.num_subcores
    b_per_w = bsz // nw
    ch = 128
    n_ch = b_per_w // ch
    nbuf = 3
    mesh = plsc.VectorSubcoreMesh(core_axis_name="c", subcore_axis_name="s")

    @functools.partial(
        pl.kernel,
        mesh=mesh,
        out_type=jax.ShapeDtypeStruct((bsz, d), jnp.int32),
        scratch_types=[
            pltpu.VMEM((b_per_w,), jnp.int32),
            *[pltpu.VMEM((ch, d), jnp.int32) for _ in range(nbuf)],
            *[pltpu.SemaphoreType.DMA for _ in range(2 * nbuf)],
        ],
    )
    def gather_k(table_hbm, idx_hbm, out_hbm, idx_v, *bufs_and_sems):
        rows = bufs_and_sems[:nbuf]
        gsem = bufs_and_sems[nbuf:2 * nbuf]
        osem = bufs_and_sems[2 * nbuf:]
        wid = lax.axis_index("s") * info.num_cores + lax.axis_index("c")
        base = wid * b_per_w
        pltpu.sync_copy(idx_hbm.at[pl.ds(base, b_per_w)], idx_v)

        gcp = [None] * n_ch
        ocp = [None] * n_ch
        for c in range(n_ch):
            b = c % nbuf
            if c >= nbuf:
                ocp[c - nbuf].wait()
            gcp[c] = pltpu.async_copy(
                table_hbm.at[idx_v.at[pl.ds(c * ch, ch)]], rows[b], gsem[b])
            if c >= 1:
                pb = (c - 1) % nbuf
                gcp[c - 1].wait()
                ocp[c - 1] = pltpu.async_copy(
                    rows[pb], out_hbm.at[pl.ds(base + (c - 1) * ch, ch)], osem[pb])
        lb = (n_ch - 1) % nbuf
        gcp[n_ch - 1].wait()
        ocp[n_ch - 1] = pltpu.async_copy(
            rows[lb], out_hbm.at[pl.ds(base + (n_ch - 1) * ch, ch)], osem[lb])
        for c in range(max(0, n_ch - nbuf), n_ch):
            ocp[c].wait()

    return gather_k(table, idx)


def _stage_b_body(v_ref, p1_ref, g1_ref, e_ref, mask_ref,
                  w1e_ref, w2_ref, b2_ref, w3_ref, b3_ref,
                  nmg_ref, nmb_ref,
                  fw1_ref, fb1_ref, fw2_ref, fb2_ref, fg_ref, fb_ref,
                  ewi_ref, ebi_ref, ewj_ref,
                  v2_ref, p2_ref, q2_ref):
    nb, k = g1_ref.shape[1], g1_ref.shape[2]
    d = e_ref.shape[3]
    e = e_ref[0].reshape(nb * k, d).astype(jnp.bfloat16)
    g1 = _unpack_bf16_pair(g1_ref[0].reshape(nb * k, d // 2))
    p1 = p1_ref[0]
    x = jnp.dot(e, w1e_ref[...], preferred_element_type=jnp.float32) + g1
    x = (x.reshape(nb, k, d) + p1[:, None, :]).reshape(nb * k, d)
    h = _gelu(x).astype(jnp.bfloat16)
    h = _gelu(jnp.dot(h, w2_ref[...], preferred_element_type=jnp.float32) + b2_ref[...]).astype(jnp.bfloat16)
    m = jnp.dot(h, w3_ref[...], preferred_element_type=jnp.float32) + b3_ref[...]
    m = m.reshape(nb, k, d) * mask_ref[0][:, :, None]
    msum = jnp.sum(m, axis=1)
    v1 = _ln(v_ref[0] + msum, nmg_ref[...], nmb_ref[...])
    f = _gelu(jnp.dot(v1.astype(jnp.bfloat16), fw1_ref[...], preferred_element_type=jnp.float32) + fb1_ref[...]).astype(jnp.bfloat16)
    f = jnp.dot(f, fw2_ref[...], preferred_element_type=jnp.float32) + fb2_ref[...]
    v2 = _ln(v1 + f, fg_ref[...], fb_ref[...])
    v2_ref[0] = v2
    v2b = v2.astype(jnp.bfloat16)
    p2_ref[0] = jnp.dot(v2b, ewi_ref[...], preferred_element_type=jnp.float32) + ebi_ref[...]
    q2_ref[0] = _pack_bf16_pair(jnp.dot(v2b, ewj_ref[...], preferred_element_type=jnp.float32))


def _stage_b(zi, V, P1, G1z, E, mask, w1e, w2, b2, w3, b3, nmg, nmb,
             fw1, fb1, fw2, fb2, fg, fb, ewi, ebi, ewj):
    z, n, d = V.shape
    k = E.shape[2]
    grid = (n // NB,)
    node_spec = pl.BlockSpec((1, NB, d), lambda ni: (zi, ni, 0))
    out_node_spec = pl.BlockSpec((1, NB, d), lambda ni: (0, ni, 0))
    out_q_spec = pl.BlockSpec((1, NB, d // 2), lambda ni: (0, ni, 0))
    g_spec = pl.BlockSpec((1, NB, k, d // 2), lambda ni: (0, ni, 0, 0))
    edge_spec = pl.BlockSpec((1, NB, k, d), lambda ni: (zi, ni, 0, 0))
    mask_spec = pl.BlockSpec((1, NB, k), lambda ni: (zi, ni, 0))

    def wspec(shape):
        return pl.BlockSpec(shape, lambda ni: tuple(0 for _ in shape))

    out = jax.ShapeDtypeStruct((1, n, d), jnp.float32)
    out_q = jax.ShapeDtypeStruct((1, n, d // 2), jnp.int32)
    return pl.pallas_call(
        _stage_b_body,
        grid=grid,
        in_specs=[
            node_spec, node_spec, g_spec, edge_spec, mask_spec,
            wspec((d, d)), wspec((d, d)), wspec((1, d)), wspec((d, d)), wspec((1, d)),
            wspec((1, d)), wspec((1, d)),
            wspec((d, 4 * d)), wspec((1, 4 * d)), wspec((4 * d, d)), wspec((1, d)),
            wspec((1, d)), wspec((1, d)),
            wspec((d, d)), wspec((1, d)), wspec((d, d)),
        ],
        out_specs=(out_node_spec, out_node_spec, out_q_spec),
        out_shape=(out, out, out_q),
    )(V, P1, G1z, E, mask, w1e, w2, b2, w3, b3, nmg, nmb,
      fw1, fb1, fw2, fb2, fg, fb, ewi, ebi, ewj)


def _stage_c_body(p2_ref, g2_ref, e_ref, mask_ref,
                  w1e_ref, w2_ref, b2_ref, w3_ref, b3_ref,
                  lg_ref, lb_ref, *chain_and_out):
    eout_ref = chain_and_out[-1]
    nb, k = g2_ref.shape[1], g2_ref.shape[2]
    d = e_ref.shape[3]
    e = e_ref[0].reshape(nb * k, d)
    g2 = _unpack_bf16_pair(g2_ref[0].reshape(nb * k, d // 2))
    p2 = p2_ref[0]
    x = jnp.dot(e.astype(jnp.bfloat16), w1e_ref[...], preferred_element_type=jnp.float32) + g2
    x = (x.reshape(nb, k, d) + p2[:, None, :]).reshape(nb * k, d)
    h = _gelu(x).astype(jnp.bfloat16)
    h = _gelu(jnp.dot(h, w2_ref[...], preferred_element_type=jnp.float32) + b2_ref[...]).astype(jnp.bfloat16)
    m = jnp.dot(h, w3_ref[...], preferred_element_type=jnp.float32) + b3_ref[...]
    m = m.reshape(nb, k, d) * mask_ref[0][:, :, None]
    eout_ref[0] = _ln(e.reshape(nb, k, d) + m, lg_ref[...], lb_ref[...])


def _stage_c(zi, chain, P2z, G2z, E, mask, w1e, w2, b2, w3, b3, lg, lb):
    z, n, k, d = E.shape
    grid = (n // NB,)
    node_spec = pl.BlockSpec((1, NB, d), lambda ni: (0, ni, 0))
    g_spec = pl.BlockSpec((1, NB, k, d // 2), lambda ni: (0, ni, 0, 0))
    edge_spec = pl.BlockSpec((1, NB, k, d), lambda ni: (zi, ni, 0, 0))
    mask_spec = pl.BlockSpec((1, NB, k), lambda ni: (zi, ni, 0))

    def wspec(shape):
        return pl.BlockSpec(shape, lambda ni: tuple(0 for _ in shape))

    in_specs = [
        node_spec, g_spec, edge_spec, mask_spec,
        wspec((d, d)), wspec((d, d)), wspec((1, d)), wspec((d, d)), wspec((1, d)),
        wspec((1, d)), wspec((1, d)),
    ]
    args = [P2z, G2z, E, mask, w1e, w2, b2, w3, b3, lg, lb]
    aliases = {}
    if chain is not None:
        in_specs.append(pl.BlockSpec((1, 1, 8, 128), lambda ni: (0, 0, 0, 0)))
        args.append(chain)
        aliases = {11: 0}
    return pl.pallas_call(
        _stage_c_body,
        grid=grid,
        in_specs=in_specs,
        out_specs=edge_spec,
        out_shape=jax.ShapeDtypeStruct((z, n, k, d), jnp.float32),
        input_output_aliases=aliases,
    )(*args)


def kernel(V, E, K, edge_mask, nm_W1, nm_b1, nm_W2, nm_b2, nm_W3, nm_b3,
           nm_ln_g, nm_ln_b, ffn_W1, ffn_b1, ffn_W2, ffn_b2, ffn_ln_g, ffn_ln_b,
           em_W1, em_b1, em_W2, em_b2, em_W3, em_b3, em_ln_g, em_ln_b):
    z, n, d = V.shape
    k = K.shape[2]

    bf = jnp.bfloat16
    nm_w1_i, nm_w1_j, nm_w1_e = (nm_W1[:d].astype(bf), nm_W1[d:2 * d].astype(bf),
                                 nm_W1[2 * d:].astype(bf))
    em_w1_i, em_w1_j, em_w1_e = (em_W1[:d].astype(bf), em_W1[d:2 * d].astype(bf),
                                 em_W1[2 * d:].astype(bf))
    nm_W2, nm_W3 = nm_W2.astype(bf), nm_W3.astype(bf)
    em_W2, em_W3 = em_W2.astype(bf), em_W3.astype(bf)
    ffn_W1, ffn_W2 = ffn_W1.astype(bf), ffn_W2.astype(bf)

    idx_z = K.reshape(z, n * k)

    v_flat = V.reshape(z * n, d)
    P1, Q1 = _stage_a(v_flat, nm_w1_i, nm_b1, nm_w1_j)
    P1 = P1.reshape(z, n, d)
    Q1 = Q1.reshape(z, n, d // 2)

    b_args = (nm_w1_e, nm_W2, nm_b2.reshape(1, d), nm_W3, nm_b3.reshape(1, d),
              nm_ln_g.reshape(1, d), nm_ln_b.reshape(1, d),
              ffn_W1, ffn_b1.reshape(1, 4 * d), ffn_W2, ffn_b2.reshape(1, d),
              ffn_ln_g.reshape(1, d), ffn_ln_b.reshape(1, d),
              em_w1_i, em_b1.reshape(1, d), em_w1_j)
    c_args = (em_w1_e, em_W2, em_b2.reshape(1, d), em_W3, em_b3.reshape(1, d),
              em_ln_g.reshape(1, d), em_ln_b.reshape(1, d))

    G1 = [None] * z
    for zi in range(z):
        G1[zi] = _sc_gather(Q1[zi], idx_z[zi]).reshape(1, n, k, d // 2)

    V2 = [None] * z
    P2 = [None] * z
    Q2 = [None] * z
    for zi in range(z):
        V2[zi], P2[zi], Q2[zi] = _stage_b(
            zi, V, P1, G1[zi], E, edge_mask, *b_args)

    G2 = [None] * z
    for zi in range(z):
        G2[zi] = _sc_gather(Q2[zi].reshape(n, d // 2), idx_z[zi]).reshape(1, n, k, d // 2)

    chain = None
    for zi in range(z):
        chain = _stage_c(zi, chain, P2[zi], G2[zi], E, edge_mask, *c_args)

    return (jnp.concatenate(V2, axis=0), chain)


# speedup vs baseline: 1.1343x; 1.1002x over previous
import functools

import jax
import jax.numpy as jnp
from jax import lax
from jax.experimental import pallas as pl
from jax.experimental.pallas import tpu as pltpu
from jax.experimental.pallas import tpu_sc as plsc

NB = 256


def _gelu(x):
    return 0.5 * x * (1.0 + lax.erf(x * 0.7071067811865476))


def _pack_bf16_pair(q):
    h = q.shape[-1] // 2
    a = jax.lax.bitcast_convert_type(q[:, :h].astype(jnp.bfloat16), jnp.uint16)
    b = jax.lax.bitcast_convert_type(q[:, h:].astype(jnp.bfloat16), jnp.uint16)
    w = a.astype(jnp.uint32) | (b.astype(jnp.uint32) << 16)
    return jax.lax.bitcast_convert_type(w, jnp.int32)


def _unpack_bf16_pair(g):
    gu = jax.lax.bitcast_convert_type(g, jnp.uint32)
    a = jax.lax.bitcast_convert_type((gu & 0xFFFF).astype(jnp.uint16), jnp.bfloat16)
    b = jax.lax.bitcast_convert_type((gu >> 16).astype(jnp.uint16), jnp.bfloat16)
    return jnp.concatenate([a.astype(jnp.float32), b.astype(jnp.float32)], axis=-1)


def _ln(x, g, b):
    m = jnp.mean(x, axis=-1, keepdims=True)
    c = x - m
    v = jnp.mean(c * c, axis=-1, keepdims=True)
    return c * jax.lax.rsqrt(v + 1e-5) * g + b


def _stage_a_body(v_ref, wi_ref, bi_ref, wj_ref, p_ref, *q_refs):
    v = v_ref[...].astype(jnp.bfloat16)
    p_ref[...] = jnp.dot(v, wi_ref[...], preferred_element_type=jnp.float32) + bi_ref[...]
    q = jnp.dot(v, wj_ref[...], preferred_element_type=jnp.float32)
    qp = _pack_bf16_pair(q)
    nz = len(q_refs)
    n = qp.shape[0] // nz
    for i, qr in enumerate(q_refs):
        qr[...] = qp[i * n:(i + 1) * n]


def _stage_a(v_flat, w1_i, b1, w1_j, z):
    zn, d = v_flat.shape
    n = zn // z
    return pl.pallas_call(
        _stage_a_body,
        out_shape=(jax.ShapeDtypeStruct((zn, d), jnp.float32),
                   *[jax.ShapeDtypeStruct((n, d // 2), jnp.int32) for _ in range(z)]),
    )(v_flat, w1_i, b1.reshape(1, d), w1_j)


def _sc_gather(table, k2d, zi):
    n, d = table.shape
    bsz = k2d.shape[1]
    info = plsc.get_sparse_core_info()
    nw = info.num_cores * info.num_subcores
    b_per_w = bsz // nw
    ch = 128
    n_ch = b_per_w // ch
    nbuf = 3
    mesh = plsc.VectorSubcoreMesh(core_axis_name="c", subcore_axis_name="s")

    @functools.partial(
        pl.kernel,
        mesh=mesh,
        out_type=jax.ShapeDtypeStruct((bsz, d), jnp.int32),
        scratch_types=[
            pltpu.VMEM((b_per_w,), jnp.int32),
            *[pltpu.VMEM((ch, d), jnp.int32) for _ in range(nbuf)],
            pltpu.VMEM_SHARED((n, d), jnp.int32),
            *[pltpu.SemaphoreType.DMA for _ in range(2 * nbuf)],
        ],
    )
    def gather_k(table_hbm, idx_hbm, out_hbm, idx_v, *bufs_and_sems):
        rows = bufs_and_sems[:nbuf]
        spt = bufs_and_sems[nbuf]
        gsem = bufs_and_sems[nbuf + 1:nbuf + 1 + nbuf]
        osem = bufs_and_sems[nbuf + 1 + nbuf:]
        sid = lax.axis_index("s")
        wid = sid * info.num_cores + lax.axis_index("c")
        base = wid * b_per_w

        @pl.when(sid == 0)
        def _load_table():
            pltpu.sync_copy(table_hbm, spt)

        pltpu.sync_copy(idx_hbm.at[zi, pl.ds(base, b_per_w)], idx_v)
        plsc.subcore_barrier()

        gcp = [None] * n_ch
        ocp = [None] * n_ch
        for c in range(n_ch):
            b = c % nbuf
            if c >= nbuf:
                ocp[c - nbuf].wait()
            gcp[c] = pltpu.async_copy(
                spt.at[idx_v.at[pl.ds(c * ch, ch)]], rows[b], gsem[b])
            if c >= 1:
                pb = (c - 1) % nbuf
                gcp[c - 1].wait()
                ocp[c - 1] = pltpu.async_copy(
                    rows[pb], out_hbm.at[pl.ds(base + (c - 1) * ch, ch)], osem[pb])
        lb = (n_ch - 1) % nbuf
        gcp[n_ch - 1].wait()
        ocp[n_ch - 1] = pltpu.async_copy(
            rows[lb], out_hbm.at[pl.ds(base + (n_ch - 1) * ch, ch)], osem[lb])
        for c in range(max(0, n_ch - nbuf), n_ch):
            ocp[c].wait()

    return gather_k(table, k2d)


def _stage_b_body(v_ref, p1_ref, g1_ref, e_ref, mask_ref,
                  w1e_ref, w2_ref, b2_ref, w3_ref, b3_ref,
                  nmg_ref, nmb_ref,
                  fw1_ref, fb1_ref, fw2_ref, fb2_ref, fg_ref, fb_ref,
                  ewi_ref, ebi_ref, ewj_ref,
                  v2_ref, p2_ref, q2_ref):
    nb, k = g1_ref.shape[1], g1_ref.shape[2]
    d = e_ref.shape[3]
    e = e_ref[0].reshape(nb * k, d).astype(jnp.bfloat16)
    g1 = _unpack_bf16_pair(g1_ref[0].reshape(nb * k, d // 2))
    p1 = p1_ref[0]
    x = jnp.dot(e, w1e_ref[...], preferred_element_type=jnp.float32) + g1
    x = (x.reshape(nb, k, d) + p1[:, None, :]).reshape(nb * k, d)
    h = _gelu(x).astype(jnp.bfloat16)
    h = _gelu(jnp.dot(h, w2_ref[...], preferred_element_type=jnp.float32) + b2_ref[...]).astype(jnp.bfloat16)
    m = jnp.dot(h, w3_ref[...], preferred_element_type=jnp.float32) + b3_ref[...]
    m = m.reshape(nb, k, d) * mask_ref[0][:, :, None]
    msum = jnp.sum(m, axis=1)
    v1 = _ln(v_ref[0] + msum, nmg_ref[...], nmb_ref[...])
    f = _gelu(jnp.dot(v1.astype(jnp.bfloat16), fw1_ref[...], preferred_element_type=jnp.float32) + fb1_ref[...]).astype(jnp.bfloat16)
    f = jnp.dot(f, fw2_ref[...], preferred_element_type=jnp.float32) + fb2_ref[...]
    v2 = _ln(v1 + f, fg_ref[...], fb_ref[...])
    v2_ref[0] = v2
    v2b = v2.astype(jnp.bfloat16)
    p2_ref[0] = jnp.dot(v2b, ewi_ref[...], preferred_element_type=jnp.float32) + ebi_ref[...]
    q2_ref[0] = _pack_bf16_pair(jnp.dot(v2b, ewj_ref[...], preferred_element_type=jnp.float32))


def _stage_b(zi, V, P1, G1z, E, mask, w1e, w2, b2, w3, b3, nmg, nmb,
             fw1, fb1, fw2, fb2, fg, fb, ewi, ebi, ewj):
    z, n, d = V.shape
    k = E.shape[2]
    grid = (n // NB,)
    node_spec = pl.BlockSpec((1, NB, d), lambda ni: (zi, ni, 0))
    out_node_spec = pl.BlockSpec((1, NB, d), lambda ni: (0, ni, 0))
    out_q_spec = pl.BlockSpec((1, NB, d // 2), lambda ni: (0, ni, 0))
    g_spec = pl.BlockSpec((1, NB, k, d // 2), lambda ni: (0, ni, 0, 0))
    edge_spec = pl.BlockSpec((1, NB, k, d), lambda ni: (zi, ni, 0, 0))
    mask_spec = pl.BlockSpec((1, NB, k), lambda ni: (zi, ni, 0))

    def wspec(shape):
        return pl.BlockSpec(shape, lambda ni: tuple(0 for _ in shape))

    out = jax.ShapeDtypeStruct((1, n, d), jnp.float32)
    out_q = jax.ShapeDtypeStruct((1, n, d // 2), jnp.int32)
    return pl.pallas_call(
        _stage_b_body,
        grid=grid,
        in_specs=[
            node_spec, node_spec, g_spec, edge_spec, mask_spec,
            wspec((d, d)), wspec((d, d)), wspec((1, d)), wspec((d, d)), wspec((1, d)),
            wspec((1, d)), wspec((1, d)),
            wspec((d, 4 * d)), wspec((1, 4 * d)), wspec((4 * d, d)), wspec((1, d)),
            wspec((1, d)), wspec((1, d)),
            wspec((d, d)), wspec((1, d)), wspec((d, d)),
        ],
        out_specs=(out_node_spec, out_node_spec, out_q_spec),
        out_shape=(out, out, out_q),
    )(V, P1, G1z, E, mask, w1e, w2, b2, w3, b3, nmg, nmb,
      fw1, fb1, fw2, fb2, fg, fb, ewi, ebi, ewj)


def _stage_c_body(p2_ref, g2_ref, e_ref, mask_ref,
                  w1e_ref, w2_ref, b2_ref, w3_ref, b3_ref,
                  lg_ref, lb_ref, *chain_and_out):
    eout_ref = chain_and_out[-1]
    nb, k = g2_ref.shape[1], g2_ref.shape[2]
    d = e_ref.shape[3]
    e = e_ref[0].reshape(nb * k, d)
    g2 = _unpack_bf16_pair(g2_ref[0].reshape(nb * k, d // 2))
    p2 = p2_ref[0]
    x = jnp.dot(e.astype(jnp.bfloat16), w1e_ref[...], preferred_element_type=jnp.float32) + g2
    x = (x.reshape(nb, k, d) + p2[:, None, :]).reshape(nb * k, d)
    h = _gelu(x).astype(jnp.bfloat16)
    h = _gelu(jnp.dot(h, w2_ref[...], preferred_element_type=jnp.float32) + b2_ref[...]).astype(jnp.bfloat16)
    m = jnp.dot(h, w3_ref[...], preferred_element_type=jnp.float32) + b3_ref[...]
    m = m.reshape(nb, k, d) * mask_ref[0][:, :, None]
    eout_ref[0] = _ln(e.reshape(nb, k, d) + m, lg_ref[...], lb_ref[...])


def _stage_c(zi, chain, P2z, G2z, E, mask, w1e, w2, b2, w3, b3, lg, lb):
    z, n, k, d = E.shape
    grid = (n // NB,)
    node_spec = pl.BlockSpec((1, NB, d), lambda ni: (0, ni, 0))
    g_spec = pl.BlockSpec((1, NB, k, d // 2), lambda ni: (0, ni, 0, 0))
    edge_spec = pl.BlockSpec((1, NB, k, d), lambda ni: (zi, ni, 0, 0))
    mask_spec = pl.BlockSpec((1, NB, k), lambda ni: (zi, ni, 0))

    def wspec(shape):
        return pl.BlockSpec(shape, lambda ni: tuple(0 for _ in shape))

    in_specs = [
        node_spec, g_spec, edge_spec, mask_spec,
        wspec((d, d)), wspec((d, d)), wspec((1, d)), wspec((d, d)), wspec((1, d)),
        wspec((1, d)), wspec((1, d)),
    ]
    args = [P2z, G2z, E, mask, w1e, w2, b2, w3, b3, lg, lb]
    aliases = {}
    if chain is not None:
        in_specs.append(pl.BlockSpec((1, 1, 8, 128), lambda ni: (0, 0, 0, 0)))
        args.append(chain)
        aliases = {11: 0}
    return pl.pallas_call(
        _stage_c_body,
        grid=grid,
        in_specs=in_specs,
        out_specs=edge_spec,
        out_shape=jax.ShapeDtypeStruct((z, n, k, d), jnp.float32),
        input_output_aliases=aliases,
    )(*args)


def kernel(V, E, K, edge_mask, nm_W1, nm_b1, nm_W2, nm_b2, nm_W3, nm_b3,
           nm_ln_g, nm_ln_b, ffn_W1, ffn_b1, ffn_W2, ffn_b2, ffn_ln_g, ffn_ln_b,
           em_W1, em_b1, em_W2, em_b2, em_W3, em_b3, em_ln_g, em_ln_b):
    z, n, d = V.shape
    k = K.shape[2]

    bf = jnp.bfloat16
    nm_w1_i, nm_w1_j, nm_w1_e = (nm_W1[:d].astype(bf), nm_W1[d:2 * d].astype(bf),
                                 nm_W1[2 * d:].astype(bf))
    em_w1_i, em_w1_j, em_w1_e = (em_W1[:d].astype(bf), em_W1[d:2 * d].astype(bf),
                                 em_W1[2 * d:].astype(bf))
    nm_W2, nm_W3 = nm_W2.astype(bf), nm_W3.astype(bf)
    em_W2, em_W3 = em_W2.astype(bf), em_W3.astype(bf)
    ffn_W1, ffn_W2 = ffn_W1.astype(bf), ffn_W2.astype(bf)

    idx_z = K.reshape(z, n * k)

    v_flat = V.reshape(z * n, d)
    P1, *Q1s = _stage_a(v_flat, nm_w1_i, nm_b1, nm_w1_j, z)
    P1 = P1.reshape(z, n, d)

    b_args = (nm_w1_e, nm_W2, nm_b2.reshape(1, d), nm_W3, nm_b3.reshape(1, d),
              nm_ln_g.reshape(1, d), nm_ln_b.reshape(1, d),
              ffn_W1, ffn_b1.reshape(1, 4 * d), ffn_W2, ffn_b2.reshape(1, d),
              ffn_ln_g.reshape(1, d), ffn_ln_b.reshape(1, d),
              em_w1_i, em_b1.reshape(1, d), em_w1_j)
    c_args = (em_w1_e, em_W2, em_b2.reshape(1, d), em_W3, em_b3.reshape(1, d),
              em_ln_g.reshape(1, d), em_ln_b.reshape(1, d))

    G1 = [None] * z
    for zi in range(z):
        G1[zi] = _sc_gather(Q1s[zi], idx_z, zi).reshape(1, n, k, d // 2)

    V2 = [None] * z
    P2 = [None] * z
    Q2 = [None] * z
    for zi in range(z):
        V2[zi], P2[zi], Q2[zi] = _stage_b(
            zi, V, P1, G1[zi], E, edge_mask, *b_args)

    G2 = [None] * z
    for zi in range(z):
        G2[zi] = _sc_gather(Q2[zi].reshape(n, d // 2), idx_z, zi).reshape(1, n, k, d // 2)

    chain = None
    for zi in range(z):
        chain = _stage_c(zi, chain, P2[zi], G2[zi], E, edge_mask, *c_args)

    return (jnp.concatenate(V2, axis=0), chain)


# speedup vs baseline: 1.1447x; 1.0092x over previous
import functools

import jax
import jax.numpy as jnp
from jax import lax
from jax.experimental import pallas as pl
from jax.experimental.pallas import tpu as pltpu
from jax.experimental.pallas import tpu_sc as plsc

NB = 256


def _gelu(x):
    return 0.5 * x * (1.0 + lax.erf(x * 0.7071067811865476))


def _pack_bf16_pair(q):
    h = q.shape[-1] // 2
    a = jax.lax.bitcast_convert_type(q[:, :h].astype(jnp.bfloat16), jnp.uint16)
    b = jax.lax.bitcast_convert_type(q[:, h:].astype(jnp.bfloat16), jnp.uint16)
    w = a.astype(jnp.uint32) | (b.astype(jnp.uint32) << 16)
    return jax.lax.bitcast_convert_type(w, jnp.int32)


def _unpack_bf16_pair(g):
    gu = jax.lax.bitcast_convert_type(g, jnp.uint32)
    a = jax.lax.bitcast_convert_type((gu & 0xFFFF).astype(jnp.uint16), jnp.bfloat16)
    b = jax.lax.bitcast_convert_type((gu >> 16).astype(jnp.uint16), jnp.bfloat16)
    return jnp.concatenate([a.astype(jnp.float32), b.astype(jnp.float32)], axis=-1)


def _ln(x, g, b):
    m = jnp.mean(x, axis=-1, keepdims=True)
    c = x - m
    v = jnp.mean(c * c, axis=-1, keepdims=True)
    return c * jax.lax.rsqrt(v + 1e-5) * g + b


def _stage_a_body(v_ref, wi_ref, bi_ref, wj_ref, p_ref, *q_refs):
    v = v_ref[...].astype(jnp.bfloat16)
    p_ref[...] = jnp.dot(v, wi_ref[...], preferred_element_type=jnp.float32) + bi_ref[...]
    q = jnp.dot(v, wj_ref[...], preferred_element_type=jnp.float32)
    qp = _pack_bf16_pair(q)
    nz = len(q_refs)
    n = qp.shape[0] // nz
    for i, qr in enumerate(q_refs):
        qr[...] = qp[i * n:(i + 1) * n]


def _stage_a(v_flat, w1_i, b1, w1_j, z):
    zn, d = v_flat.shape
    n = zn // z
    return pl.pallas_call(
        _stage_a_body,
        out_shape=(jax.ShapeDtypeStruct((zn, d), jnp.float32),
                   *[jax.ShapeDtypeStruct((n, d // 2), jnp.int32) for _ in range(z)]),
    )(v_flat, w1_i, b1.reshape(1, d), w1_j)


def _sc_gather(table, k2d, zi):
    n, d = table.shape
    bsz = k2d.shape[1]
    info = plsc.get_sparse_core_info()
    nw = info.num_cores * info.num_subcores
    b_per_w = bsz // nw
    ch = 128
    n_ch = b_per_w // ch
    nbuf = 3
    mesh = plsc.VectorSubcoreMesh(core_axis_name="c", subcore_axis_name="s")

    @functools.partial(
        pl.kernel,
        mesh=mesh,
        out_type=jax.ShapeDtypeStruct((bsz, d), jnp.int32),
        scratch_types=[
            pltpu.VMEM((b_per_w,), jnp.int32),
            *[pltpu.VMEM((ch, d), jnp.int32) for _ in range(nbuf)],
            pltpu.VMEM_SHARED((n, d), jnp.int32),
            *[pltpu.SemaphoreType.DMA for _ in range(2 * nbuf)],
        ],
    )
    def gather_k(table_hbm, idx_hbm, out_hbm, idx_v, *bufs_and_sems):
        rows = bufs_and_sems[:nbuf]
        spt = bufs_and_sems[nbuf]
        gsem = bufs_and_sems[nbuf + 1:nbuf + 1 + nbuf]
        osem = bufs_and_sems[nbuf + 1 + nbuf:]
        sid = lax.axis_index("s")
        wid = sid * info.num_cores + lax.axis_index("c")
        base = wid * b_per_w

        rps = n // info.num_subcores
        pltpu.sync_copy(table_hbm.at[pl.ds(sid * rps, rps)], rows[0].at[pl.ds(0, rps)])
        pltpu.sync_copy(rows[0].at[pl.ds(0, rps)], spt.at[pl.ds(sid * rps, rps)])
        pltpu.sync_copy(idx_hbm.at[zi, pl.ds(base, b_per_w)], idx_v)
        plsc.subcore_barrier()

        gcp = [None] * n_ch
        ocp = [None] * n_ch
        for c in range(n_ch):
            b = c % nbuf
            if c >= nbuf:
                ocp[c - nbuf].wait()
            gcp[c] = pltpu.async_copy(
                spt.at[idx_v.at[pl.ds(c * ch, ch)]], rows[b], gsem[b])
            if c >= 1:
                pb = (c - 1) % nbuf
                gcp[c - 1].wait()
                ocp[c - 1] = pltpu.async_copy(
                    rows[pb], out_hbm.at[pl.ds(base + (c - 1) * ch, ch)], osem[pb])
        lb = (n_ch - 1) % nbuf
        gcp[n_ch - 1].wait()
        ocp[n_ch - 1] = pltpu.async_copy(
            rows[lb], out_hbm.at[pl.ds(base + (n_ch - 1) * ch, ch)], osem[lb])
        for c in range(max(0, n_ch - nbuf), n_ch):
            ocp[c].wait()

    return gather_k(table, k2d)


def _stage_b_body(v_ref, p1_ref, g1_ref, e_ref, mask_ref,
                  w1e_ref, w2_ref, b2_ref, w3_ref, b3_ref,
                  nmg_ref, nmb_ref,
                  fw1_ref, fb1_ref, fw2_ref, fb2_ref, fg_ref, fb_ref,
                  ewi_ref, ebi_ref, ewj_ref,
                  v2_ref, p2_ref, q2_ref):
    nb, k = g1_ref.shape[1], g1_ref.shape[2]
    d = e_ref.shape[3]
    e = e_ref[0].reshape(nb * k, d).astype(jnp.bfloat16)
    g1 = _unpack_bf16_pair(g1_ref[0].reshape(nb * k, d // 2))
    p1 = p1_ref[0]
    x = jnp.dot(e, w1e_ref[...], preferred_element_type=jnp.float32) + g1
    x = (x.reshape(nb, k, d) + p1[:, None, :]).reshape(nb * k, d)
    h = _gelu(x).astype(jnp.bfloat16)
    h = _gelu(jnp.dot(h, w2_ref[...], preferred_element_type=jnp.float32) + b2_ref[...]).astype(jnp.bfloat16)
    m = jnp.dot(h, w3_ref[...], preferred_element_type=jnp.float32) + b3_ref[...]
    m = m.reshape(nb, k, d) * mask_ref[0][:, :, None]
    msum = jnp.sum(m, axis=1)
    v1 = _ln(v_ref[0] + msum, nmg_ref[...], nmb_ref[...])
    f = _gelu(jnp.dot(v1.astype(jnp.bfloat16), fw1_ref[...], preferred_element_type=jnp.float32) + fb1_ref[...]).astype(jnp.bfloat16)
    f = jnp.dot(f, fw2_ref[...], preferred_element_type=jnp.float32) + fb2_ref[...]
    v2 = _ln(v1 + f, fg_ref[...], fb_ref[...])
    v2_ref[0] = v2
    v2b = v2.astype(jnp.bfloat16)
    p2_ref[0] = jnp.dot(v2b, ewi_ref[...], preferred_element_type=jnp.float32) + ebi_ref[...]
    q2_ref[0] = _pack_bf16_pair(jnp.dot(v2b, ewj_ref[...], preferred_element_type=jnp.float32))


def _stage_b(zi, V, P1, G1z, E, mask, w1e, w2, b2, w3, b3, nmg, nmb,
             fw1, fb1, fw2, fb2, fg, fb, ewi, ebi, ewj):
    z, n, d = V.shape
    k = E.shape[2]
    grid = (n // NB,)
    node_spec = pl.BlockSpec((1, NB, d), lambda ni: (zi, ni, 0))
    out_node_spec = pl.BlockSpec((1, NB, d), lambda ni: (0, ni, 0))
    out_q_spec = pl.BlockSpec((1, NB, d // 2), lambda ni: (0, ni, 0))
    g_spec = pl.BlockSpec((1, NB, k, d // 2), lambda ni: (0, ni, 0, 0))
    edge_spec = pl.BlockSpec((1, NB, k, d), lambda ni: (zi, ni, 0, 0))
    mask_spec = pl.BlockSpec((1, NB, k), lambda ni: (zi, ni, 0))

    def wspec(shape):
        return pl.BlockSpec(shape, lambda ni: tuple(0 for _ in shape))

    out = jax.ShapeDtypeStruct((1, n, d), jnp.float32)
    out_q = jax.ShapeDtypeStruct((1, n, d // 2), jnp.int32)
    return pl.pallas_call(
        _stage_b_body,
        grid=grid,
        in_specs=[
            node_spec, node_spec, g_spec, edge_spec, mask_spec,
            wspec((d, d)), wspec((d, d)), wspec((1, d)), wspec((d, d)), wspec((1, d)),
            wspec((1, d)), wspec((1, d)),
            wspec((d, 4 * d)), wspec((1, 4 * d)), wspec((4 * d, d)), wspec((1, d)),
            wspec((1, d)), wspec((1, d)),
            wspec((d, d)), wspec((1, d)), wspec((d, d)),
        ],
        out_specs=(out_node_spec, out_node_spec, out_q_spec),
        out_shape=(out, out, out_q),
    )(V, P1, G1z, E, mask, w1e, w2, b2, w3, b3, nmg, nmb,
      fw1, fb1, fw2, fb2, fg, fb, ewi, ebi, ewj)


def _stage_c_body(p2_ref, g2_ref, e_ref, mask_ref,
                  w1e_ref, w2_ref, b2_ref, w3_ref, b3_ref,
                  lg_ref, lb_ref, *chain_and_out):
    eout_ref = chain_and_out[-1]
    nb, k = g2_ref.shape[1], g2_ref.shape[2]
    d = e_ref.shape[3]
    e = e_ref[0].reshape(nb * k, d)
    g2 = _unpack_bf16_pair(g2_ref[0].reshape(nb * k, d // 2))
    p2 = p2_ref[0]
    x = jnp.dot(e.astype(jnp.bfloat16), w1e_ref[...], preferred_element_type=jnp.float32) + g2
    x = (x.reshape(nb, k, d) + p2[:, None, :]).reshape(nb * k, d)
    h = _gelu(x).astype(jnp.bfloat16)
    h = _gelu(jnp.dot(h, w2_ref[...], preferred_element_type=jnp.float32) + b2_ref[...]).astype(jnp.bfloat16)
    m = jnp.dot(h, w3_ref[...], preferred_element_type=jnp.float32) + b3_ref[...]
    m = m.reshape(nb, k, d) * mask_ref[0][:, :, None]
    eout_ref[0] = _ln(e.reshape(nb, k, d) + m, lg_ref[...], lb_ref[...])


def _stage_c(zi, chain, P2z, G2z, E, mask, w1e, w2, b2, w3, b3, lg, lb):
    z, n, k, d = E.shape
    grid = (n // NB,)
    node_spec = pl.BlockSpec((1, NB, d), lambda ni: (0, ni, 0))
    g_spec = pl.BlockSpec((1, NB, k, d // 2), lambda ni: (0, ni, 0, 0))
    edge_spec = pl.BlockSpec((1, NB, k, d), lambda ni: (zi, ni, 0, 0))
    mask_spec = pl.BlockSpec((1, NB, k), lambda ni: (zi, ni, 0))

    def wspec(shape):
        return pl.BlockSpec(shape, lambda ni: tuple(0 for _ in shape))

    in_specs = [
        node_spec, g_spec, edge_spec, mask_spec,
        wspec((d, d)), wspec((d, d)), wspec((1, d)), wspec((d, d)), wspec((1, d)),
        wspec((1, d)), wspec((1, d)),
    ]
    args = [P2z, G2z, E, mask, w1e, w2, b2, w3, b3, lg, lb]
    aliases = {}
    if chain is not None:
        in_specs.append(pl.BlockSpec((1, 1, 8, 128), lambda ni: (0, 0, 0, 0)))
        args.append(chain)
        aliases = {11: 0}
    return pl.pallas_call(
        _stage_c_body,
        grid=grid,
        in_specs=in_specs,
        out_specs=edge_spec,
        out_shape=jax.ShapeDtypeStruct((z, n, k, d), jnp.float32),
        input_output_aliases=aliases,
    )(*args)


def kernel(V, E, K, edge_mask, nm_W1, nm_b1, nm_W2, nm_b2, nm_W3, nm_b3,
           nm_ln_g, nm_ln_b, ffn_W1, ffn_b1, ffn_W2, ffn_b2, ffn_ln_g, ffn_ln_b,
           em_W1, em_b1, em_W2, em_b2, em_W3, em_b3, em_ln_g, em_ln_b):
    z, n, d = V.shape
    k = K.shape[2]

    bf = jnp.bfloat16
    nm_w1_i, nm_w1_j, nm_w1_e = (nm_W1[:d].astype(bf), nm_W1[d:2 * d].astype(bf),
                                 nm_W1[2 * d:].astype(bf))
    em_w1_i, em_w1_j, em_w1_e = (em_W1[:d].astype(bf), em_W1[d:2 * d].astype(bf),
                                 em_W1[2 * d:].astype(bf))
    nm_W2, nm_W3 = nm_W2.astype(bf), nm_W3.astype(bf)
    em_W2, em_W3 = em_W2.astype(bf), em_W3.astype(bf)
    ffn_W1, ffn_W2 = ffn_W1.astype(bf), ffn_W2.astype(bf)

    idx_z = K.reshape(z, n * k)

    v_flat = V.reshape(z * n, d)
    P1, *Q1s = _stage_a(v_flat, nm_w1_i, nm_b1, nm_w1_j, z)
    P1 = P1.reshape(z, n, d)

    b_args = (nm_w1_e, nm_W2, nm_b2.reshape(1, d), nm_W3, nm_b3.reshape(1, d),
              nm_ln_g.reshape(1, d), nm_ln_b.reshape(1, d),
              ffn_W1, ffn_b1.reshape(1, 4 * d), ffn_W2, ffn_b2.reshape(1, d),
              ffn_ln_g.reshape(1, d), ffn_ln_b.reshape(1, d),
              em_w1_i, em_b1.reshape(1, d), em_w1_j)
    c_args = (em_w1_e, em_W2, em_b2.reshape(1, d), em_W3, em_b3.reshape(1, d),
              em_ln_g.reshape(1, d), em_ln_b.reshape(1, d))

    G1 = [None] * z
    for zi in range(z):
        G1[zi] = _sc_gather(Q1s[zi], idx_z, zi).reshape(1, n, k, d // 2)

    V2 = [None] * z
    P2 = [None] * z
    Q2 = [None] * z
    for zi in range(z):
        V2[zi], P2[zi], Q2[zi] = _stage_b(
            zi, V, P1, G1[zi], E, edge_mask, *b_args)

    G2 = [None] * z
    for zi in range(z):
        G2[zi] = _sc_gather(Q2[zi].reshape(n, d // 2), idx_z, zi).reshape(1, n, k, d // 2)

    chain = None
    for zi in range(z):
        chain = _stage_c(zi, chain, P2[zi], G2[zi], E, edge_mask, *c_args)

    return (jnp.concatenate(V2, axis=0), chain)


# speedup vs baseline: 1.1759x; 1.0272x over previous
import functools

import jax
import jax.numpy as jnp
from jax import lax
from jax.experimental import pallas as pl
from jax.experimental.pallas import tpu as pltpu
from jax.experimental.pallas import tpu_sc as plsc

NB = 256


def _gelu(x):
    return 0.5 * x * (1.0 + lax.erf(x * 0.7071067811865476))


def _pack_bf16_pair(q):
    h = q.shape[-1] // 2
    a = jax.lax.bitcast_convert_type(q[:, :h].astype(jnp.bfloat16), jnp.uint16)
    b = jax.lax.bitcast_convert_type(q[:, h:].astype(jnp.bfloat16), jnp.uint16)
    w = a.astype(jnp.uint32) | (b.astype(jnp.uint32) << 16)
    return jax.lax.bitcast_convert_type(w, jnp.int32)


def _unpack_bf16_pair(g):
    gu = jax.lax.bitcast_convert_type(g, jnp.uint32)
    a = jax.lax.bitcast_convert_type((gu & 0xFFFF).astype(jnp.uint16), jnp.bfloat16)
    b = jax.lax.bitcast_convert_type((gu >> 16).astype(jnp.uint16), jnp.bfloat16)
    return jnp.concatenate([a.astype(jnp.float32), b.astype(jnp.float32)], axis=-1)


def _ln(x, g, b):
    m = jnp.mean(x, axis=-1, keepdims=True)
    c = x - m
    v = jnp.mean(c * c, axis=-1, keepdims=True)
    return c * jax.lax.rsqrt(v + 1e-5) * g + b


def _stage_a_body(v_ref, wj_ref, *q_refs):
    v = v_ref[...].astype(jnp.bfloat16)
    q = jnp.dot(v, wj_ref[...], preferred_element_type=jnp.float32)
    qp = _pack_bf16_pair(q)
    nz = len(q_refs)
    n = qp.shape[0] // nz
    for i, qr in enumerate(q_refs):
        qr[...] = qp[i * n:(i + 1) * n]


def _stage_a(v_flat, w1_j, z):
    zn, d = v_flat.shape
    n = zn // z
    return pl.pallas_call(
        _stage_a_body,
        out_shape=tuple(
            jax.ShapeDtypeStruct((n, d // 2), jnp.int32) for _ in range(z)),
    )(v_flat, w1_j)


def _sc_gather(table, k2d, zi):
    n, d = table.shape
    bsz = k2d.shape[1]
    info = plsc.get_sparse_core_info()
    nw = info.num_cores * info.num_subcores
    b_per_w = bsz // nw
    ch = 128
    n_ch = b_per_w // ch
    nbuf = 3
    mesh = plsc.VectorSubcoreMesh(core_axis_name="c", subcore_axis_name="s")

    @functools.partial(
        pl.kernel,
        mesh=mesh,
        out_type=jax.ShapeDtypeStruct((bsz, d), jnp.int32),
        scratch_types=[
            pltpu.VMEM((b_per_w,), jnp.int32),
            *[pltpu.VMEM((ch, d), jnp.int32) for _ in range(nbuf)],
            pltpu.VMEM_SHARED((n, d), jnp.int32),
            *[pltpu.SemaphoreType.DMA for _ in range(2 * nbuf)],
        ],
    )
    def gather_k(table_hbm, idx_hbm, out_hbm, idx_v, *bufs_and_sems):
        rows = bufs_and_sems[:nbuf]
        spt = bufs_and_sems[nbuf]
        gsem = bufs_and_sems[nbuf + 1:nbuf + 1 + nbuf]
        osem = bufs_and_sems[nbuf + 1 + nbuf:]
        sid = lax.axis_index("s")
        wid = sid * info.num_cores + lax.axis_index("c")
        base = wid * b_per_w

        rps = n // info.num_subcores
        pltpu.sync_copy(table_hbm.at[pl.ds(sid * rps, rps)], rows[0].at[pl.ds(0, rps)])
        pltpu.sync_copy(rows[0].at[pl.ds(0, rps)], spt.at[pl.ds(sid * rps, rps)])
        pltpu.sync_copy(idx_hbm.at[zi, pl.ds(base, b_per_w)], idx_v)
        plsc.subcore_barrier()

        gcp = [None] * n_ch
        ocp = [None] * n_ch
        for c in range(n_ch):
            b = c % nbuf
            if c >= nbuf:
                ocp[c - nbuf].wait()
            gcp[c] = pltpu.async_copy(
                spt.at[idx_v.at[pl.ds(c * ch, ch)]], rows[b], gsem[b])
            if c >= 1:
                pb = (c - 1) % nbuf
                gcp[c - 1].wait()
                ocp[c - 1] = pltpu.async_copy(
                    rows[pb], out_hbm.at[pl.ds(base + (c - 1) * ch, ch)], osem[pb])
        lb = (n_ch - 1) % nbuf
        gcp[n_ch - 1].wait()
        ocp[n_ch - 1] = pltpu.async_copy(
            rows[lb], out_hbm.at[pl.ds(base + (n_ch - 1) * ch, ch)], osem[lb])
        for c in range(max(0, n_ch - nbuf), n_ch):
            ocp[c].wait()

    return gather_k(table, k2d)


def _stage_b_body(v_ref, g1_ref, e_ref, mask_ref,
                  w1i_ref, b1_ref, w1e_ref, w2_ref, b2_ref, w3_ref, b3_ref,
                  nmg_ref, nmb_ref,
                  fw1_ref, fb1_ref, fw2_ref, fb2_ref, fg_ref, fb_ref,
                  ewi_ref, ebi_ref, ewj_ref,
                  v2_ref, p2_ref, q2_ref):
    nb, k = g1_ref.shape[1], g1_ref.shape[2]
    d = e_ref.shape[3]
    e = e_ref[0].reshape(nb * k, d).astype(jnp.bfloat16)
    g1 = _unpack_bf16_pair(g1_ref[0].reshape(nb * k, d // 2))
    p1 = jnp.dot(v_ref[0].astype(jnp.bfloat16), w1i_ref[...],
                 preferred_element_type=jnp.float32) + b1_ref[...]
    x = jnp.dot(e, w1e_ref[...], preferred_element_type=jnp.float32) + g1
    x = (x.reshape(nb, k, d) + p1[:, None, :]).reshape(nb * k, d)
    h = _gelu(x).astype(jnp.bfloat16)
    h = _gelu(jnp.dot(h, w2_ref[...], preferred_element_type=jnp.float32) + b2_ref[...]).astype(jnp.bfloat16)
    m = jnp.dot(h, w3_ref[...], preferred_element_type=jnp.float32) + b3_ref[...]
    m = m.reshape(nb, k, d) * mask_ref[0][:, :, None]
    msum = jnp.sum(m, axis=1)
    v1 = _ln(v_ref[0] + msum, nmg_ref[...], nmb_ref[...])
    f = _gelu(jnp.dot(v1.astype(jnp.bfloat16), fw1_ref[...], preferred_element_type=jnp.float32) + fb1_ref[...]).astype(jnp.bfloat16)
    f = jnp.dot(f, fw2_ref[...], preferred_element_type=jnp.float32) + fb2_ref[...]
    v2 = _ln(v1 + f, fg_ref[...], fb_ref[...])
    v2_ref[0] = v2
    v2b = v2.astype(jnp.bfloat16)
    p2_ref[0] = jnp.dot(v2b, ewi_ref[...], preferred_element_type=jnp.float32) + ebi_ref[...]
    q2_ref[0] = _pack_bf16_pair(jnp.dot(v2b, ewj_ref[...], preferred_element_type=jnp.float32))


def _stage_b(zi, V, G1z, E, mask, w1i, b1, w1e, w2, b2, w3, b3, nmg, nmb,
             fw1, fb1, fw2, fb2, fg, fb, ewi, ebi, ewj):
    z, n, d = V.shape
    k = E.shape[2]
    grid = (n // NB,)
    node_spec = pl.BlockSpec((1, NB, d), lambda ni: (zi, ni, 0))
    out_node_spec = pl.BlockSpec((1, NB, d), lambda ni: (0, ni, 0))
    out_q_spec = pl.BlockSpec((1, NB, d // 2), lambda ni: (0, ni, 0))
    g_spec = pl.BlockSpec((1, NB, k, d // 2), lambda ni: (0, ni, 0, 0))
    edge_spec = pl.BlockSpec((1, NB, k, d), lambda ni: (zi, ni, 0, 0))
    mask_spec = pl.BlockSpec((1, NB, k), lambda ni: (zi, ni, 0))

    def wspec(shape):
        return pl.BlockSpec(shape, lambda ni: tuple(0 for _ in shape))

    out = jax.ShapeDtypeStruct((1, n, d), jnp.float32)
    out_q = jax.ShapeDtypeStruct((1, n, d // 2), jnp.int32)
    return pl.pallas_call(
        _stage_b_body,
        grid=grid,
        in_specs=[
            node_spec, g_spec, edge_spec, mask_spec,
            wspec((d, d)), wspec((1, d)),
            wspec((d, d)), wspec((d, d)), wspec((1, d)), wspec((d, d)), wspec((1, d)),
            wspec((1, d)), wspec((1, d)),
            wspec((d, 4 * d)), wspec((1, 4 * d)), wspec((4 * d, d)), wspec((1, d)),
            wspec((1, d)), wspec((1, d)),
            wspec((d, d)), wspec((1, d)), wspec((d, d)),
        ],
        out_specs=(out_node_spec, out_node_spec, out_q_spec),
        out_shape=(out, out, out_q),
    )(V, G1z, E, mask, w1i, b1, w1e, w2, b2, w3, b3, nmg, nmb,
      fw1, fb1, fw2, fb2, fg, fb, ewi, ebi, ewj)


def _stage_c_body(p2_ref, g2_ref, e_ref, mask_ref,
                  w1e_ref, w2_ref, b2_ref, w3_ref, b3_ref,
                  lg_ref, lb_ref, *chain_and_out):
    eout_ref = chain_and_out[-1]
    nb, k = g2_ref.shape[1], g2_ref.shape[2]
    d = e_ref.shape[3]
    e = e_ref[0].reshape(nb * k, d)
    g2 = _unpack_bf16_pair(g2_ref[0].reshape(nb * k, d // 2))
    p2 = p2_ref[0]
    x = jnp.dot(e.astype(jnp.bfloat16), w1e_ref[...], preferred_element_type=jnp.float32) + g2
    x = (x.reshape(nb, k, d) + p2[:, None, :]).reshape(nb * k, d)
    h = _gelu(x).astype(jnp.bfloat16)
    h = _gelu(jnp.dot(h, w2_ref[...], preferred_element_type=jnp.float32) + b2_ref[...]).astype(jnp.bfloat16)
    m = jnp.dot(h, w3_ref[...], preferred_element_type=jnp.float32) + b3_ref[...]
    m = m.reshape(nb, k, d) * mask_ref[0][:, :, None]
    eout_ref[0] = _ln(e.reshape(nb, k, d) + m, lg_ref[...], lb_ref[...])


def _stage_c(zi, chain, P2z, G2z, E, mask, w1e, w2, b2, w3, b3, lg, lb):
    z, n, k, d = E.shape
    grid = (n // NB,)
    node_spec = pl.BlockSpec((1, NB, d), lambda ni: (0, ni, 0))
    g_spec = pl.BlockSpec((1, NB, k, d // 2), lambda ni: (0, ni, 0, 0))
    edge_spec = pl.BlockSpec((1, NB, k, d), lambda ni: (zi, ni, 0, 0))
    mask_spec = pl.BlockSpec((1, NB, k), lambda ni: (zi, ni, 0))

    def wspec(shape):
        return pl.BlockSpec(shape, lambda ni: tuple(0 for _ in shape))

    in_specs = [
        node_spec, g_spec, edge_spec, mask_spec,
        wspec((d, d)), wspec((d, d)), wspec((1, d)), wspec((d, d)), wspec((1, d)),
        wspec((1, d)), wspec((1, d)),
    ]
    args = [P2z, G2z, E, mask, w1e, w2, b2, w3, b3, lg, lb]
    aliases = {}
    if chain is not None:
        in_specs.append(pl.BlockSpec((1, 1, 8, 128), lambda ni: (0, 0, 0, 0)))
        args.append(chain)
        aliases = {11: 0}
    return pl.pallas_call(
        _stage_c_body,
        grid=grid,
        in_specs=in_specs,
        out_specs=edge_spec,
        out_shape=jax.ShapeDtypeStruct((z, n, k, d), jnp.float32),
        input_output_aliases=aliases,
    )(*args)


def kernel(V, E, K, edge_mask, nm_W1, nm_b1, nm_W2, nm_b2, nm_W3, nm_b3,
           nm_ln_g, nm_ln_b, ffn_W1, ffn_b1, ffn_W2, ffn_b2, ffn_ln_g, ffn_ln_b,
           em_W1, em_b1, em_W2, em_b2, em_W3, em_b3, em_ln_g, em_ln_b):
    z, n, d = V.shape
    k = K.shape[2]

    bf = jnp.bfloat16
    nm_w1_i, nm_w1_j, nm_w1_e = (nm_W1[:d].astype(bf), nm_W1[d:2 * d].astype(bf),
                                 nm_W1[2 * d:].astype(bf))
    em_w1_i, em_w1_j, em_w1_e = (em_W1[:d].astype(bf), em_W1[d:2 * d].astype(bf),
                                 em_W1[2 * d:].astype(bf))
    nm_W2, nm_W3 = nm_W2.astype(bf), nm_W3.astype(bf)
    em_W2, em_W3 = em_W2.astype(bf), em_W3.astype(bf)
    ffn_W1, ffn_W2 = ffn_W1.astype(bf), ffn_W2.astype(bf)

    idx_z = K.reshape(z, n * k)

    v_flat = V.reshape(z * n, d)
    Q1s = _stage_a(v_flat, nm_w1_j, z)

    b_args = (nm_w1_i, nm_b1.reshape(1, d),
              nm_w1_e, nm_W2, nm_b2.reshape(1, d), nm_W3, nm_b3.reshape(1, d),
              nm_ln_g.reshape(1, d), nm_ln_b.reshape(1, d),
              ffn_W1, ffn_b1.reshape(1, 4 * d), ffn_W2, ffn_b2.reshape(1, d),
              ffn_ln_g.reshape(1, d), ffn_ln_b.reshape(1, d),
              em_w1_i, em_b1.reshape(1, d), em_w1_j)
    c_args = (em_w1_e, em_W2, em_b2.reshape(1, d), em_W3, em_b3.reshape(1, d),
              em_ln_g.reshape(1, d), em_ln_b.reshape(1, d))

    G1 = [None] * z
    for zi in range(z):
        G1[zi] = _sc_gather(Q1s[zi], idx_z, zi).reshape(1, n, k, d // 2)

    V2 = [None] * z
    P2 = [None] * z
    Q2 = [None] * z
    for zi in range(z):
        V2[zi], P2[zi], Q2[zi] = _stage_b(
            zi, V, G1[zi], E, edge_mask, *b_args)

    G2 = [None] * z
    for zi in range(z):
        G2[zi] = _sc_gather(Q2[zi].reshape(n, d // 2), idx_z, zi).reshape(1, n, k, d // 2)

    chain = None
    for zi in range(z):
        chain = _stage_c(zi, chain, P2[zi], G2[zi], E, edge_mask, *c_args)

    return (jnp.concatenate(V2, axis=0), chain)
